# baseline (speedup 1.0000x reference)
import functools

import jax
import jax.numpy as jnp
from jax import lax
from jax.experimental import pallas as pl
from jax.experimental.pallas import tpu as pltpu
from jax.experimental.pallas import tpu_sc as plsc

EMBED = 128
BATCH = 4
SEQ = 8192
N_IDX = BATCH * SEQ
NW = 32
B_PER_W = N_IDX // NW
W_PER_ROW = SEQ // B_PER_W
CHUNK = 128
N_CHUNKS = B_PER_W // CHUNK
DEPTH = 3
TABLE_ROWS = 8192
STAGE_ROWS = TABLE_ROWS // 16


@functools.partial(
    pl.kernel,
    out_type=jax.ShapeDtypeStruct((BATCH, SEQ, EMBED), jnp.float32),
    mesh=plsc.VectorSubcoreMesh(core_axis_name="c", subcore_axis_name="s"),
    scratch_types=[
        pltpu.VMEM((B_PER_W,), jnp.int32),
        pltpu.VMEM((DEPTH, CHUNK, EMBED), jnp.float32),
        pltpu.VMEM_SHARED((TABLE_ROWS, EMBED), jnp.float32),
        pltpu.SemaphoreType.DMA,
        pltpu.SemaphoreType.DMA,
    ],
)
def _gather_kernel(table_hbm, idx_hbm, out_hbm, idx_v, rows_v, table_sh, sem_g, sem_s):
    info = plsc.get_sparse_core_info()
    wid = lax.axis_index("s") * info.num_cores + lax.axis_index("c")
    sid = lax.axis_index("s")
    row = wid // W_PER_ROW
    col = (wid % W_PER_ROW) * B_PER_W

    pltpu.sync_copy(idx_hbm.at[row, pl.ds(col, B_PER_W)], idx_v)

    def gather(j):
        src = table_hbm if j < N_CHUNKS // 2 else table_sh
        return pltpu.async_copy(
            src.at[idx_v.at[pl.ds(j * CHUNK, CHUNK)]],
            rows_v.at[j % DEPTH],
            sem_g,
        )

    def store(j):
        return pltpu.async_copy(
            rows_v.at[j % DEPTH],
            out_hbm.at[row, pl.ds(col + j * CHUNK, CHUNK)],
            sem_s,
        )

    g = [None] * N_CHUNKS
    s = [None] * N_CHUNKS
    for j in range(min(DEPTH, N_CHUNKS)):
        g[j] = gather(j)
    pltpu.sync_copy(
        table_hbm.at[pl.ds(sid * STAGE_ROWS, STAGE_ROWS)],
        table_sh.at[pl.ds(sid * STAGE_ROWS, STAGE_ROWS)],
    )
    plsc.subcore_barrier()
    for j in range(N_CHUNKS):
        g[j].wait()
        s[j] = store(j)
        k = j + 1 - DEPTH
        if k >= 0 and k + DEPTH < N_CHUNKS:
            s[k].wait()
            g[k + DEPTH] = gather(k + DEPTH)
    for j in range(max(0, N_CHUNKS - DEPTH), N_CHUNKS):
        s[j].wait()


def kernel(time, pe):
    return _gather_kernel(pe, time.astype(jnp.int32))


# speedup vs baseline: 1.0479x; 1.0479x over previous
import functools

import jax
import jax.numpy as jnp
from jax import lax
from jax.experimental import pallas as pl
from jax.experimental.pallas import tpu as pltpu
from jax.experimental.pallas import tpu_sc as plsc

EMBED = 128
BATCH = 4
SEQ = 8192
N_IDX = BATCH * SEQ
NW = 32
B_PER_W = N_IDX // NW
W_PER_ROW = SEQ // B_PER_W
CHUNK = 128
N_CHUNKS = B_PER_W // CHUNK
DEPTH = 3
TABLE_ROWS = 8192
STAGE_ROWS = TABLE_ROWS // 16


@functools.partial(
    pl.kernel,
    out_type=jax.ShapeDtypeStruct((BATCH, SEQ, EMBED), jnp.float32),
    mesh=plsc.VectorSubcoreMesh(core_axis_name="c", subcore_axis_name="s"),
    scratch_types=[
        pltpu.VMEM((B_PER_W,), jnp.int32),
        pltpu.VMEM((DEPTH, CHUNK, EMBED), jnp.float32),
        pltpu.VMEM_SHARED((TABLE_ROWS, EMBED), jnp.float32),
        pltpu.SemaphoreType.DMA,
        pltpu.SemaphoreType.DMA,
        pltpu.SemaphoreType.DMA,
    ],
)
def _gather_kernel(table_hbm, idx_hbm, out_hbm, idx_v, rows_v, table_sh, sem_g, sem_s, sem_t):
    info = plsc.get_sparse_core_info()
    wid = lax.axis_index("s") * info.num_cores + lax.axis_index("c")
    sid = lax.axis_index("s")
    row = wid // W_PER_ROW
    col = (wid % W_PER_ROW) * B_PER_W

    stage = pltpu.async_copy(
        table_hbm.at[pl.ds(sid * STAGE_ROWS, STAGE_ROWS)],
        table_sh.at[pl.ds(sid * STAGE_ROWS, STAGE_ROWS)],
        sem_t,
    )

    pltpu.sync_copy(idx_hbm.at[row, pl.ds(col, B_PER_W)], idx_v)

    def gather(j):
        return pltpu.async_copy(
            table_sh.at[idx_v.at[pl.ds(j * CHUNK, CHUNK)]],
            rows_v.at[j % DEPTH],
            sem_g,
        )

    def store(j):
        return pltpu.async_copy(
            rows_v.at[j % DEPTH],
            out_hbm.at[row, pl.ds(col + j * CHUNK, CHUNK)],
            sem_s,
        )

    stage.wait()
    plsc.subcore_barrier()
    g = [None] * N_CHUNKS
    s = [None] * N_CHUNKS
    for j in range(min(DEPTH, N_CHUNKS)):
        g[j] = gather(j)
    for j in range(N_CHUNKS):
        g[j].wait()
        s[j] = store(j)
        k = j + 1 - DEPTH
        if k >= 0 and k + DEPTH < N_CHUNKS:
            s[k].wait()
            g[k + DEPTH] = gather(k + DEPTH)
    for j in range(max(0, N_CHUNKS - DEPTH), N_CHUNKS):
        s[j].wait()


def kernel(time, pe):
    return _gather_kernel(pe, time.astype(jnp.int32))


# speedup vs baseline: 1.0698x; 1.0209x over previous
import functools

import jax
import jax.numpy as jnp
from jax import lax
from jax.experimental import pallas as pl
from jax.experimental.pallas import tpu as pltpu
from jax.experimental.pallas import tpu_sc as plsc

EMBED = 128
BATCH = 4
SEQ = 8192
N_IDX = BATCH * SEQ
NW = 32
B_PER_W = N_IDX // NW
W_PER_ROW = SEQ // B_PER_W
CHUNK = 128
N_CHUNKS = B_PER_W // CHUNK
DEPTH = 3
TABLE_ROWS = 8192
STAGE_ROWS = TABLE_ROWS // 16


@functools.partial(
    pl.kernel,
    out_type=jax.ShapeDtypeStruct((BATCH, SEQ, EMBED), jnp.float32),
    mesh=plsc.VectorSubcoreMesh(core_axis_name="c", subcore_axis_name="s"),
    scratch_types=[
        pltpu.VMEM((B_PER_W,), jnp.int32),
        pltpu.VMEM((DEPTH, CHUNK, EMBED), jnp.float32),
        pltpu.VMEM_SHARED((TABLE_ROWS, EMBED), jnp.float32),
        pltpu.SemaphoreType.DMA,
        pltpu.SemaphoreType.DMA,
        pltpu.SemaphoreType.DMA,
    ],
)
def _gather_kernel(table_hbm, idx_hbm, out_hbm, idx_v, rows_v, table_sh, sem_g, sem_s, sem_t):
    info = plsc.get_sparse_core_info()
    wid = lax.axis_index("s") * info.num_cores + lax.axis_index("c")
    sid = lax.axis_index("s")
    row = wid // W_PER_ROW
    col = (wid % W_PER_ROW) * B_PER_W

    stage = pltpu.async_copy(
        table_hbm.at[pl.ds(sid * STAGE_ROWS, STAGE_ROWS)],
        table_sh.at[pl.ds(sid * STAGE_ROWS, STAGE_ROWS)],
        sem_t,
    )

    pltpu.sync_copy(idx_hbm.at[row, pl.ds(col, B_PER_W)], idx_v)

    def gather(j):
        src = table_hbm if j < DEPTH else table_sh
        return pltpu.async_copy(
            src.at[idx_v.at[pl.ds(j * CHUNK, CHUNK)]],
            rows_v.at[j % DEPTH],
            sem_g,
        )

    def store(j):
        return pltpu.async_copy(
            rows_v.at[j % DEPTH],
            out_hbm.at[row, pl.ds(col + j * CHUNK, CHUNK)],
            sem_s,
        )

    g = [None] * N_CHUNKS
    s = [None] * N_CHUNKS
    for j in range(min(DEPTH, N_CHUNKS)):
        g[j] = gather(j)
    stage.wait()
    plsc.subcore_barrier()
    for j in range(N_CHUNKS):
        g[j].wait()
        s[j] = store(j)
        k = j + 1 - DEPTH
        if k >= 0 and k + DEPTH < N_CHUNKS:
            s[k].wait()
            g[k + DEPTH] = gather(k + DEPTH)
    for j in range(max(0, N_CHUNKS - DEPTH), N_CHUNKS):
        s[j].wait()


def kernel(time, pe):
    return _gather_kernel(pe, time.astype(jnp.int32))
